# baseline (speedup 1.0000x reference)
import functools

import jax
import jax.numpy as jnp
from jax import lax
from jax.experimental import pallas as pl
from jax.experimental.pallas import tpu as pltpu
from jax.experimental.pallas import tpu_sc as plsc

NUM_WORKERS = 32


def kernel(x, tables):
    b, n_fields = x.shape
    _, cardp1, d = tables.shape
    assert n_fields == 26 and d == 32

    cb = 8
    bpw = b // NUM_WORKERS
    n_chunks = bpw // cb

    mesh = plsc.VectorSubcoreMesh(core_axis_name="c", subcore_axis_name="s")

    @functools.partial(
        pl.kernel,
        mesh=mesh,
        out_type=jax.ShapeDtypeStruct((b, n_fields, d), jnp.float32),
        scratch_types=[
            pltpu.VMEM((2, cb, n_fields), jnp.int32),
            pltpu.VMEM((2, cb, n_fields, d), jnp.float32),
            pltpu.SemaphoreType.DMA,
            pltpu.SemaphoreType.DMA,
            pltpu.SemaphoreType.DMA,
        ],
    )
    def emb_kernel(x_hbm, tab_hbm, out_hbm, xv, rows, sem_g, sem_x, sem_o):
        wid = lax.axis_index("s") * 2 + lax.axis_index("c")
        wb = wid * bpw

        pltpu.async_copy(x_hbm.at[pl.ds(wb, cb)], xv.at[0], sem_x)

        def chunk_body(c, carry):
            buf = c % 2
            b0 = wb + c * cb
            pltpu.make_async_copy(
                x_hbm.at[pl.ds(b0, cb)], xv.at[buf], sem_x
            ).wait()

            @pl.when(c >= 2)
            def _():
                pltpu.make_async_copy(
                    rows.at[buf],
                    out_hbm.at[pl.ds(b0 - 2 * cb, cb)],
                    sem_o,
                ).wait()

            def fire_body(kb, carry2):
                va = xv[buf, kb, pl.ds(0, 16)]
                vb = xv[buf, kb, pl.ds(n_fields - 16, 16)]
                for f in range(n_fields):
                    r = va[f] if f < 16 else vb[f - (n_fields - 16)]
                    pltpu.async_copy(
                        tab_hbm.at[f, r],
                        rows.at[buf, kb, f],
                        sem_g,
                    )
                return carry2

            lax.fori_loop(0, cb, fire_body, 0)

            @pl.when(c + 1 < n_chunks)
            def _():
                pltpu.async_copy(
                    x_hbm.at[pl.ds(b0 + cb, cb)], xv.at[1 - buf], sem_x
                )

            def drain_body(kb, carry2):
                for f in range(n_fields):
                    pltpu.make_async_copy(
                        tab_hbm.at[0, 0],
                        rows.at[buf, kb, f],
                        sem_g,
                    ).wait()
                return carry2

            lax.fori_loop(0, cb, drain_body, 0)

            pltpu.async_copy(rows.at[buf], out_hbm.at[pl.ds(b0, cb)], sem_o)
            return carry

        lax.fori_loop(0, n_chunks, chunk_body, 0)

        for t in (2, 1):
            pltpu.make_async_copy(
                rows.at[(n_chunks - t) % 2],
                out_hbm.at[pl.ds(wb + (n_chunks - t) * cb, cb)],
                sem_o,
            ).wait()

    return emb_kernel(x, tables)
